# baseline (speedup 1.0000x reference)
import jax
import jax.numpy as jnp
from jax.experimental import pallas as pl
from jax.experimental.pallas import tpu as pltpu

E = 8
TOP_K = 2
D_MODEL = 768
D_FF = 384
T = 2048
BLK_R = 512


def _router_block(x_ref, gate_ref, wf_ref, x16_ref):
    xb = x_ref[...]
    x16_ref[...] = xb.astype(jnp.bfloat16)
    logits = jax.lax.dot_general(
        xb, gate_ref[...], (((1,), (1,)), ((), ())),
        preferred_element_type=jnp.float32)
    m = jnp.max(logits, axis=1, keepdims=True)
    ex = jnp.exp(logits - m)
    s = ex / jnp.sum(ex, axis=1, keepdims=True)
    idx = jax.lax.broadcasted_iota(jnp.int32, (BLK_R, E), 1)
    v1 = jnp.max(s, axis=1, keepdims=True)
    i1 = jnp.min(jnp.where(s == v1, idx, E), axis=1, keepdims=True)
    s2 = jnp.where(idx == i1, -jnp.inf, s)
    v2 = jnp.max(s2, axis=1, keepdims=True)
    i2 = jnp.min(jnp.where(s2 == v2, idx, E), axis=1, keepdims=True)
    denom = v1 + v2
    wf_ref[...] = (jnp.where(idx == i1, v1 / denom, 0.0)
                   + jnp.where(idx == i2, v2 / denom, 0.0))


def _router(x, gate_w):
    return pl.pallas_call(
        _router_block,
        grid=(T // BLK_R,),
        in_specs=[
            pl.BlockSpec((BLK_R, D_MODEL), lambda i: (i, 0)),
            pl.BlockSpec((E, D_MODEL), lambda i: (0, 0)),
        ],
        out_specs=[
            pl.BlockSpec((BLK_R, E), lambda i: (i, 0)),
            pl.BlockSpec((BLK_R, D_MODEL), lambda i: (i, 0)),
        ],
        out_shape=[
            jax.ShapeDtypeStruct((T, E), jnp.float32),
            jax.ShapeDtypeStruct((T, D_MODEL), jnp.bfloat16),
        ],
    )(x, gate_w)


def _moe_step(x16_ref, wf_ref, wg_ref, wu_ref, wd_ref, y_ref):
    e = pl.program_id(0)
    xb16 = x16_ref[...]
    g = jax.lax.dot_general(
        xb16, wg_ref[0].astype(jnp.bfloat16), (((1,), (1,)), ((), ())),
        preferred_element_type=jnp.float32)
    u = jax.lax.dot_general(
        xb16, wu_ref[0].astype(jnp.bfloat16), (((1,), (1,)), ((), ())),
        preferred_element_type=jnp.float32)
    eidx = jax.lax.broadcasted_iota(jnp.int32, (T, E), 1)
    we = jnp.sum(jnp.where(eidx == e, wf_ref[...], 0.0), axis=1,
                 keepdims=True)
    h = (g / (1.0 + jnp.exp(-g))) * u * we
    o = jax.lax.dot_general(
        h.astype(jnp.bfloat16), wd_ref[0].astype(jnp.bfloat16),
        (((1,), (1,)), ((), ())),
        preferred_element_type=jnp.float32)

    @pl.when(e == 0)
    def _init():
        y_ref[...] = o

    @pl.when(e > 0)
    def _acc():
        y_ref[...] += o


@jax.jit
def _moe(x, gate_w, W_gate, W_up, W_down):
    wf, x16 = _router(x, gate_w)
    return pl.pallas_call(
        _moe_step,
        grid=(E,),
        in_specs=[
            pl.BlockSpec((T, D_MODEL), lambda e: (0, 0)),
            pl.BlockSpec((T, E), lambda e: (0, 0)),
            pl.BlockSpec((1, D_FF, D_MODEL), lambda e: (e, 0, 0)),
            pl.BlockSpec((1, D_FF, D_MODEL), lambda e: (e, 0, 0)),
            pl.BlockSpec((1, D_MODEL, D_FF), lambda e: (e, 0, 0)),
        ],
        out_specs=pl.BlockSpec((T, D_MODEL), lambda e: (0, 0)),
        out_shape=jax.ShapeDtypeStruct((T, D_MODEL), jnp.float32),
    )(x16, wf, W_gate, W_up, W_down)


def kernel(hidden_states, gate_w, W_gate, W_up, W_down):
    orig_shape = hidden_states.shape
    x = hidden_states.reshape(-1, orig_shape[-1])
    y = _moe(x, gate_w, W_gate, W_up, W_down)
    return y.reshape(orig_shape)


# speedup vs baseline: 1.1381x; 1.1381x over previous
import jax
import jax.numpy as jnp
from jax.experimental import pallas as pl
from jax.experimental.pallas import tpu as pltpu

E = 8
TOP_K = 2
D_MODEL = 768
D_FF = 384
T = 2048


def _moe_step(x_ref, gate_ref, wg_ref, wu_ref, wd_ref, y_ref, wf_ref,
              x16_ref):
    e = pl.program_id(0)

    @pl.when(e == 0)
    def _router():
        xb = x_ref[...]
        logits = jax.lax.dot_general(
            xb, gate_ref[...], (((1,), (1,)), ((), ())),
            preferred_element_type=jnp.float32)
        m = jnp.max(logits, axis=1, keepdims=True)
        ex = jnp.exp(logits - m)
        s = ex / jnp.sum(ex, axis=1, keepdims=True)
        idx = jax.lax.broadcasted_iota(jnp.int32, (T, E), 1)
        v1 = jnp.max(s, axis=1, keepdims=True)
        i1 = jnp.min(jnp.where(s == v1, idx, E), axis=1, keepdims=True)
        s2 = jnp.where(idx == i1, -jnp.inf, s)
        v2 = jnp.max(s2, axis=1, keepdims=True)
        i2 = jnp.min(jnp.where(s2 == v2, idx, E), axis=1, keepdims=True)
        denom = v1 + v2
        wf_ref[...] = (jnp.where(idx == i1, v1 / denom, 0.0)
                       + jnp.where(idx == i2, v2 / denom, 0.0))
        y_ref[...] = jnp.zeros((T, D_MODEL), jnp.float32)
        x16_ref[...] = xb.astype(jnp.bfloat16)

    xb16 = x16_ref[...]
    g = jax.lax.dot_general(
        xb16, wg_ref[0].astype(jnp.bfloat16), (((1,), (1,)), ((), ())),
        preferred_element_type=jnp.float32)
    u = jax.lax.dot_general(
        xb16, wu_ref[0].astype(jnp.bfloat16), (((1,), (1,)), ((), ())),
        preferred_element_type=jnp.float32)
    eidx = jax.lax.broadcasted_iota(jnp.int32, (T, E), 1)
    we = jnp.sum(jnp.where(eidx == e, wf_ref[...], 0.0), axis=1,
                 keepdims=True)
    h = (g / (1.0 + jnp.exp(-g))) * u * we
    o = jax.lax.dot_general(
        h.astype(jnp.bfloat16), wd_ref[0].astype(jnp.bfloat16),
        (((1,), (1,)), ((), ())),
        preferred_element_type=jnp.float32)
    y_ref[...] += o


@jax.jit
def _moe(x, gate_w, W_gate, W_up, W_down):
    return pl.pallas_call(
        _moe_step,
        grid=(E,),
        in_specs=[
            pl.BlockSpec((T, D_MODEL), lambda e: (0, 0)),
            pl.BlockSpec((E, D_MODEL), lambda e: (0, 0)),
            pl.BlockSpec((1, D_FF, D_MODEL), lambda e: (e, 0, 0)),
            pl.BlockSpec((1, D_FF, D_MODEL), lambda e: (e, 0, 0)),
            pl.BlockSpec((1, D_MODEL, D_FF), lambda e: (e, 0, 0)),
        ],
        out_specs=pl.BlockSpec((T, D_MODEL), lambda e: (0, 0)),
        out_shape=jax.ShapeDtypeStruct((T, D_MODEL), jnp.float32),
        scratch_shapes=[pltpu.VMEM((T, E), jnp.float32),
                        pltpu.VMEM((T, D_MODEL), jnp.bfloat16)],
    )(x, gate_w, W_gate, W_up, W_down)


def kernel(hidden_states, gate_w, W_gate, W_up, W_down):
    orig_shape = hidden_states.shape
    x = hidden_states.reshape(-1, orig_shape[-1])
    y = _moe(x, gate_w, W_gate, W_up, W_down)
    return y.reshape(orig_shape)
